# Initial kernel scaffold; baseline (speedup 1.0000x reference)
import jax
import jax.numpy as jnp
from jax.experimental import pallas as pl


def kernel(x, edge_index, W1, b1, Wmu, bmu, Wsig, bsig, Wnc, bnc, Wnf, bnf):
    raise NotImplementedError("write your pallas kernel here")


# speedup vs baseline: 11.9781x; 11.9781x over previous
import functools

import jax
import jax.numpy as jnp
from jax import lax
from jax.experimental import pallas as pl
from jax.experimental.pallas import tpu as pltpu
from jax.experimental.pallas import tpu_sc as plsc

_NC = 2
_NS = 16
_CH = 128


def _seg_sum(n, e, w, gather):
    nchunks = e // _CH
    half = nchunks // _NC
    iters = -(-half // _NS)
    rpt = (n // _NS) & ~7
    last = n - (_NS - 1) * rpt

    def _striped_copy(mk_src, mk_dst, sid):
        @pl.when(sid < _NS - 1)
        def _():
            pltpu.sync_copy(mk_src(sid * rpt, rpt), mk_dst(sid * rpt, rpt))

        @pl.when(sid == _NS - 1)
        def _():
            pltpu.sync_copy(mk_src((_NS - 1) * rpt, last),
                            mk_dst((_NS - 1) * rpt, last))

    def body(src_hbm, dst_hbm, table_hbm, zeros_hbm, out_hbm,
             sidx, didx, rows, acc, sem):
        cid = lax.axis_index("c")
        sid = lax.axis_index("s")
        _striped_copy(lambda o, s: zeros_hbm.at[pl.ds(o, s)],
                      lambda o, s: acc.at[pl.ds(o, s)], sid)
        if not gather:
            pltpu.sync_copy(table_hbm, rows)
        plsc.subcore_barrier()

        def step(k, carry):
            ch = k * _NS + sid

            @pl.when(ch < half)
            def _():
                off = (cid * half + ch) * _CH
                pltpu.sync_copy(dst_hbm.at[pl.ds(off, _CH)], didx)
                if gather:
                    pltpu.sync_copy(src_hbm.at[pl.ds(off, _CH)], sidx)
                    pltpu.async_copy(table_hbm.at[sidx], rows, sem).wait()
                pltpu.sync_copy(rows, acc.at[didx], add=True)

            return carry

        lax.fori_loop(0, iters, step, 0)
        plsc.subcore_barrier()
        _striped_copy(lambda o, s: acc.at[pl.ds(o, s)],
                      lambda o, s: out_hbm.at[pl.ds(cid * n + o, s)], sid)

    return functools.partial(
        pl.kernel,
        out_type=jax.ShapeDtypeStruct((_NC * n, w), jnp.float32),
        mesh=plsc.VectorSubcoreMesh(core_axis_name="c", subcore_axis_name="s"),
        compiler_params=pltpu.CompilerParams(use_tc_tiling_on_sc=False)
        if w < 128 else None,
        scratch_types=[
            pltpu.VMEM((_CH,), jnp.int32),
            pltpu.VMEM((_CH,), jnp.int32),
            pltpu.VMEM((_CH, w), jnp.float32),
            pltpu.VMEM_SHARED((n, w), jnp.float32),
            pltpu.SemaphoreType.DMA,
        ])(body)


def _dinv_of(da_ref, db_ref):
    deg = da_ref[:, 0:1] + db_ref[:, 0:1] + 1.0
    return lax.rsqrt(deg)


def _tc_prescale(n, bn):
    grid = n // bn

    def body(x_ref, w_ref, da_ref, db_ref, o_ref):
        dinv = _dinv_of(da_ref, db_ref)
        o_ref[...] = jnp.dot(x_ref[...], w_ref[...],
                             preferred_element_type=jnp.float32) * dinv

    return pl.pallas_call(
        body,
        grid=(grid,),
        in_specs=[
            pl.BlockSpec((bn, 128), lambda i: (i, 0)),
            pl.BlockSpec((128, 128), lambda i: (0, 0)),
            pl.BlockSpec((bn, 16), lambda i: (i, 0)),
            pl.BlockSpec((bn, 16), lambda i: (i + grid, 0)),
        ],
        out_specs=pl.BlockSpec((bn, 128), lambda i: (i, 0)),
        out_shape=jax.ShapeDtypeStruct((n, 128), jnp.float32),
    )


def _tc_mid(n, bn):
    grid = n // bn

    def body(pa_ref, pb_ref, hs_ref, da_ref, db_ref, b_ref, w_ref, o_ref):
        dinv = _dinv_of(da_ref, db_ref)
        h = (pa_ref[...] + pb_ref[...] + hs_ref[...]) * dinv + b_ref[...]
        h = jnp.maximum(h, 0.0)
        o_ref[...] = jnp.dot(h, w_ref[...],
                             preferred_element_type=jnp.float32) * dinv

    return pl.pallas_call(
        body,
        grid=(grid,),
        in_specs=[
            pl.BlockSpec((bn, 128), lambda i: (i, 0)),
            pl.BlockSpec((bn, 128), lambda i: (i + grid, 0)),
            pl.BlockSpec((bn, 128), lambda i: (i, 0)),
            pl.BlockSpec((bn, 16), lambda i: (i, 0)),
            pl.BlockSpec((bn, 16), lambda i: (i + grid, 0)),
            pl.BlockSpec((1, 128), lambda i: (0, 0)),
            pl.BlockSpec((128, 64), lambda i: (0, 0)),
        ],
        out_specs=pl.BlockSpec((bn, 64), lambda i: (i, 0)),
        out_shape=jax.ShapeDtypeStruct((n, 64), jnp.float32),
    )


def _tc_decode(n, bn, dnc, dnf):
    grid = n // bn

    def body(pa_ref, pb_ref, hs_ref, da_ref, db_ref, bmu_ref,
             wnc_ref, bnc_ref, wnf_ref, bnf_ref, f_ref, ft_ref):
        dinv = _dinv_of(da_ref, db_ref)
        mu = (pa_ref[...] + pb_ref[...] + hs_ref[...]) * dinv + bmu_ref[...]
        f_ref[...] = jnp.dot(mu, wnc_ref[...],
                             preferred_element_type=jnp.float32) + bnc_ref[...]
        ft_ref[...] = jnp.dot(mu, wnf_ref[...],
                              preferred_element_type=jnp.float32) + bnf_ref[...]

    return pl.pallas_call(
        body,
        grid=(grid,),
        in_specs=[
            pl.BlockSpec((bn, 64), lambda i: (i, 0)),
            pl.BlockSpec((bn, 64), lambda i: (i + grid, 0)),
            pl.BlockSpec((bn, 64), lambda i: (i, 0)),
            pl.BlockSpec((bn, 16), lambda i: (i, 0)),
            pl.BlockSpec((bn, 16), lambda i: (i + grid, 0)),
            pl.BlockSpec((1, 64), lambda i: (0, 0)),
            pl.BlockSpec((64, dnc), lambda i: (0, 0)),
            pl.BlockSpec((1, dnc), lambda i: (0, 0)),
            pl.BlockSpec((64, dnf), lambda i: (0, 0)),
            pl.BlockSpec((1, dnf), lambda i: (0, 0)),
        ],
        out_specs=[
            pl.BlockSpec((bn, dnc), lambda i: (i, 0)),
            pl.BlockSpec((bn, dnf), lambda i: (i, 0)),
        ],
        out_shape=[
            jax.ShapeDtypeStruct((n, dnc), jnp.float32),
            jax.ShapeDtypeStruct((n, dnf), jnp.float32),
        ],
    )


def kernel(x, edge_index, W1, b1, Wmu, bmu, Wsig, bsig, Wnc, bnc, Wnf, bnf):
    n = x.shape[0]
    e = edge_index.shape[1]
    hid = Wmu.shape[1]
    dnc = Wnc.shape[1]
    dnf = Wnf.shape[1]
    nmax = 40
    bn = 1000

    src = edge_index[0]
    dst = edge_index[1]

    ones16 = jnp.ones((_CH, 16), jnp.float32)
    z16 = jnp.zeros((n, 16), jnp.float32)
    z128 = jnp.zeros((n, 2 * hid), jnp.float32)
    z64 = jnp.zeros((n, hid), jnp.float32)

    dega = _seg_sum(n, e, 16, gather=False)(dst, dst, ones16, z16)
    hs1 = _tc_prescale(n, bn)(x, W1, dega, dega)
    p1 = _seg_sum(n, e, 2 * hid, gather=True)(src, dst, hs1, z128)
    hs2 = _tc_mid(n, bn)(p1, p1, hs1, dega, dega,
                         b1.reshape(1, -1), Wmu)
    p2 = _seg_sum(n, e, hid, gather=True)(src, dst, hs2, z64)
    f, ft = _tc_decode(n, bn, dnc, dnf)(p2, p2, hs2, dega, dega,
                                        bmu.reshape(1, -1),
                                        Wnc, bnc.reshape(1, -1),
                                        Wnf, bnf.reshape(1, -1))
    return (f.reshape(n, nmax, dnc // nmax),
            ft.reshape(n, nmax, dnf // nmax))


# speedup vs baseline: 16.2902x; 1.3600x over previous
import functools

import jax
import jax.numpy as jnp
from jax import lax
from jax.experimental import pallas as pl
from jax.experimental.pallas import tpu as pltpu
from jax.experimental.pallas import tpu_sc as plsc

_NC = 2
_NS = 16
_CH = 128


_NW = _NC * _NS


def _chunk_layout(e):
    nchunks = e // _CH
    bmax = (-(-nchunks // _NW) + 7) & ~7
    return nchunks, bmax


def _striped(n):
    rpt = (n // _NS) & ~7
    last = n - (_NS - 1) * rpt

    def copy(mk_src, mk_dst, sid):
        @pl.when(sid < _NS - 1)
        def _():
            pltpu.sync_copy(mk_src(sid * rpt, rpt), mk_dst(sid * rpt, rpt))

        @pl.when(sid == _NS - 1)
        def _():
            pltpu.sync_copy(mk_src((_NS - 1) * rpt, last),
                            mk_dst((_NS - 1) * rpt, last))

    return copy


def _seg_sum(n, e, w, nb=2, phases=1):
    nchunks, bmax = _chunk_layout(e)
    striped = _striped(n)
    pb = bmax // phases
    assert pb % nb == 0 and pb * phases == bmax

    def body(src_hbm, dst_hbm, table_hbm, zeros_hbm, out_hbm,
             sidx_all, didx_all, *rest):
        rows = rest[:nb]
        sg = rest[nb:2 * nb]
        acc = rest[2 * nb]
        cid = lax.axis_index("c")
        sid = lax.axis_index("s")
        wid = cid * _NS + sid
        w0 = wid * bmax
        nch = jnp.minimum(bmax, nchunks - w0)

        striped(lambda o, s: zeros_hbm.at[pl.ds(o, s)],
                lambda o, s: acc.at[pl.ds(o, s)], sid)
        plsc.subcore_barrier()

        for ph in range(phases):
            nch_ph = jnp.clip(nch - ph * pb, 0, pb)
            pltpu.sync_copy(src_hbm.at[pl.ds(w0 + ph * pb, pb)], sidx_all)
            pltpu.sync_copy(dst_hbm.at[pl.ds(w0 + ph * pb, pb)], didx_all)

            def step(i, carry):
                base = i * nb
                gds = [None] * nb
                for b in range(nb):
                    k = base + b

                    @pl.when(k < nch_ph)
                    def _(b=b, k=k):
                        gds[b] = pltpu.make_async_copy(
                            table_hbm.at[sidx_all.at[k]], rows[b], sg[b])
                        gds[b].start()
                for b in range(nb):
                    k = base + b

                    @pl.when(k < nch_ph)
                    def _(b=b, k=k):
                        gds[b].wait()
                        pltpu.sync_copy(rows[b], acc.at[didx_all.at[k]],
                                        add=True)
                return carry

            lax.fori_loop(0, pb // nb, step, 0)

        plsc.subcore_barrier()
        striped(lambda o, s: acc.at[pl.ds(o, s)],
                lambda o, s: out_hbm.at[pl.ds(cid * n + o, s)], sid)

    return functools.partial(
        pl.kernel,
        out_type=jax.ShapeDtypeStruct((_NC * n, w), jnp.float32),
        mesh=plsc.VectorSubcoreMesh(core_axis_name="c", subcore_axis_name="s"),
        compiler_params=pltpu.CompilerParams(use_tc_tiling_on_sc=False)
        if w < 128 else None,
        scratch_types=[
            pltpu.VMEM((pb, _CH), jnp.int32),
            pltpu.VMEM((pb, _CH), jnp.int32),
        ] + [pltpu.VMEM((_CH, w), jnp.float32)] * nb
          + [pltpu.SemaphoreType.DMA] * nb
          + [pltpu.VMEM_SHARED((n, w), jnp.float32)],
    )(body)


def _deg_count(n, e, grp=4):
    nchunks, bmax = _chunk_layout(e)
    striped = _striped(n)
    assert bmax % grp == 0

    def body(dst_hbm, ones_hbm, zeros_hbm, out_hbm, didx_all, rows, *rest):
        sems = rest[:grp]
        acc = rest[grp]
        cid = lax.axis_index("c")
        sid = lax.axis_index("s")
        wid = cid * _NS + sid
        w0 = wid * bmax
        nch = jnp.minimum(bmax, nchunks - w0)

        pltpu.sync_copy(dst_hbm.at[pl.ds(w0, bmax)], didx_all)
        pltpu.sync_copy(ones_hbm, rows)
        striped(lambda o, s: zeros_hbm.at[pl.ds(o, s)],
                lambda o, s: acc.at[pl.ds(o, s)], sid)
        plsc.subcore_barrier()

        def step(i, carry):
            base = i * grp
            sds = [None] * grp
            for b in range(grp):
                k = base + b

                @pl.when(k < nch)
                def _(b=b, k=k):
                    sds[b] = pltpu.make_async_copy(
                        rows, acc.at[didx_all.at[k]], sems[b])
                    sds[b].start(add=True)
            for b in range(grp):
                k = base + b

                @pl.when(k < nch)
                def _(b=b):
                    sds[b].wait()
            return carry

        lax.fori_loop(0, bmax // grp, step, 0)
        plsc.subcore_barrier()
        striped(lambda o, s: acc.at[pl.ds(o, s)],
                lambda o, s: out_hbm.at[pl.ds(cid * n + o, s)], sid)

    return functools.partial(
        pl.kernel,
        out_type=jax.ShapeDtypeStruct((_NC * n, 16), jnp.float32),
        mesh=plsc.VectorSubcoreMesh(core_axis_name="c", subcore_axis_name="s"),
        scratch_types=[
            pltpu.VMEM((bmax, _CH), jnp.int32),
            pltpu.VMEM((_CH, 16), jnp.float32),
        ] + [pltpu.SemaphoreType.DMA] * grp
          + [pltpu.VMEM_SHARED((n, 16), jnp.float32)],
    )(body)


def _dinv_of(da_ref, db_ref):
    deg = da_ref[:, 0:1] + db_ref[:, 0:1] + 1.0
    return lax.rsqrt(deg)


def _tc_prescale(n, bn):
    grid = n // bn

    def body(x_ref, w_ref, da_ref, db_ref, o_ref):
        dinv = _dinv_of(da_ref, db_ref)
        o_ref[...] = jnp.dot(x_ref[...], w_ref[...],
                             preferred_element_type=jnp.float32) * dinv

    return pl.pallas_call(
        body,
        grid=(grid,),
        in_specs=[
            pl.BlockSpec((bn, 128), lambda i: (i, 0)),
            pl.BlockSpec((128, 128), lambda i: (0, 0)),
            pl.BlockSpec((bn, 16), lambda i: (i, 0)),
            pl.BlockSpec((bn, 16), lambda i: (i + grid, 0)),
        ],
        out_specs=pl.BlockSpec((bn, 128), lambda i: (i, 0)),
        out_shape=jax.ShapeDtypeStruct((n, 128), jnp.float32),
    )


def _tc_mid(n, bn):
    grid = n // bn

    def body(pa_ref, pb_ref, hs_ref, da_ref, db_ref, b_ref, w_ref, o_ref):
        dinv = _dinv_of(da_ref, db_ref)
        h = (pa_ref[...] + pb_ref[...] + hs_ref[...]) * dinv + b_ref[...]
        h = jnp.maximum(h, 0.0)
        o_ref[...] = jnp.dot(h, w_ref[...],
                             preferred_element_type=jnp.float32) * dinv

    return pl.pallas_call(
        body,
        grid=(grid,),
        in_specs=[
            pl.BlockSpec((bn, 128), lambda i: (i, 0)),
            pl.BlockSpec((bn, 128), lambda i: (i + grid, 0)),
            pl.BlockSpec((bn, 128), lambda i: (i, 0)),
            pl.BlockSpec((bn, 16), lambda i: (i, 0)),
            pl.BlockSpec((bn, 16), lambda i: (i + grid, 0)),
            pl.BlockSpec((1, 128), lambda i: (0, 0)),
            pl.BlockSpec((128, 64), lambda i: (0, 0)),
        ],
        out_specs=pl.BlockSpec((bn, 64), lambda i: (i, 0)),
        out_shape=jax.ShapeDtypeStruct((n, 64), jnp.float32),
    )


def _tc_decode(n, bn, dnc, dnf):
    grid = n // bn

    def body(pa_ref, pb_ref, hs_ref, da_ref, db_ref, bmu_ref,
             wnc_ref, bnc_ref, wnf_ref, bnf_ref, f_ref, ft_ref):
        dinv = _dinv_of(da_ref, db_ref)
        mu = (pa_ref[...] + pb_ref[...] + hs_ref[...]) * dinv + bmu_ref[...]
        f_ref[...] = jnp.dot(mu, wnc_ref[...],
                             preferred_element_type=jnp.float32) + bnc_ref[...]
        ft_ref[...] = jnp.dot(mu, wnf_ref[...],
                              preferred_element_type=jnp.float32) + bnf_ref[...]

    return pl.pallas_call(
        body,
        grid=(grid,),
        in_specs=[
            pl.BlockSpec((bn, 64), lambda i: (i, 0)),
            pl.BlockSpec((bn, 64), lambda i: (i + grid, 0)),
            pl.BlockSpec((bn, 64), lambda i: (i, 0)),
            pl.BlockSpec((bn, 16), lambda i: (i, 0)),
            pl.BlockSpec((bn, 16), lambda i: (i + grid, 0)),
            pl.BlockSpec((1, 64), lambda i: (0, 0)),
            pl.BlockSpec((64, dnc), lambda i: (0, 0)),
            pl.BlockSpec((1, dnc), lambda i: (0, 0)),
            pl.BlockSpec((64, dnf), lambda i: (0, 0)),
            pl.BlockSpec((1, dnf), lambda i: (0, 0)),
        ],
        out_specs=[
            pl.BlockSpec((bn, dnc), lambda i: (i, 0)),
            pl.BlockSpec((bn, dnf), lambda i: (i, 0)),
        ],
        out_shape=[
            jax.ShapeDtypeStruct((n, dnc), jnp.float32),
            jax.ShapeDtypeStruct((n, dnf), jnp.float32),
        ],
    )


def kernel(x, edge_index, W1, b1, Wmu, bmu, Wsig, bsig, Wnc, bnc, Wnf, bnf):
    n = x.shape[0]
    e = edge_index.shape[1]
    hid = Wmu.shape[1]
    dnc = Wnc.shape[1]
    dnf = Wnf.shape[1]
    nmax = 40
    bn = 1000

    _, bmax = _chunk_layout(e)
    pad = bmax * _NW * _CH - e
    src2d = jnp.pad(edge_index[0], (0, pad)).reshape(-1, _CH)
    dst2d = jnp.pad(edge_index[1], (0, pad)).reshape(-1, _CH)

    ones16 = jnp.ones((_CH, 16), jnp.float32)
    z16 = jnp.zeros((n, 16), jnp.float32)
    z128 = jnp.zeros((n, 2 * hid), jnp.float32)
    z64 = jnp.zeros((n, hid), jnp.float32)

    dega = _deg_count(n, e)(dst2d, ones16, z16)
    hs1 = _tc_prescale(n, bn)(x, W1, dega, dega)
    p1 = _seg_sum(n, e, 2 * hid, phases=2)(src2d, dst2d, hs1, z128)
    hs2 = _tc_mid(n, bn)(p1, p1, hs1, dega, dega,
                         b1.reshape(1, -1), Wmu)
    p2 = _seg_sum(n, e, hid)(src2d, dst2d, hs2, z64)
    f, ft = _tc_decode(n, bn, dnc, dnf)(p2, p2, hs2, dega, dega,
                                        bmu.reshape(1, -1),
                                        Wnc, bnc.reshape(1, -1),
                                        Wnf, bnf.reshape(1, -1))
    return (f.reshape(n, nmax, dnc // nmax),
            ft.reshape(n, nmax, dnf // nmax))


# speedup vs baseline: 17.3191x; 1.0632x over previous
import functools

import jax
import jax.numpy as jnp
from jax import lax
from jax.experimental import pallas as pl
from jax.experimental.pallas import tpu as pltpu
from jax.experimental.pallas import tpu_sc as plsc

_NC = 2
_NS = 16
_CH = 128


_NW = _NC * _NS


def _chunk_layout(e):
    nchunks = e // _CH
    bmax = (-(-nchunks // _NW) + 7) & ~7
    return nchunks, bmax


def _striped(n):
    rpt = (n // _NS) & ~7
    last = n - (_NS - 1) * rpt

    def copy(mk_src, mk_dst, sid):
        @pl.when(sid < _NS - 1)
        def _():
            pltpu.sync_copy(mk_src(sid * rpt, rpt), mk_dst(sid * rpt, rpt))

        @pl.when(sid == _NS - 1)
        def _():
            pltpu.sync_copy(mk_src((_NS - 1) * rpt, last),
                            mk_dst((_NS - 1) * rpt, last))

    return copy


def _seg_sum(n, e, w, nb=2, phases=1):
    nchunks, bmax = _chunk_layout(e)
    striped = _striped(n)
    pb = bmax // phases
    assert pb % nb == 0 and pb * phases == bmax

    def body(src_hbm, dst_hbm, table_hbm, zeros_hbm, out_hbm,
             sidx_all, didx_all, *rest):
        rows = rest[:nb]
        sg = rest[nb:2 * nb]
        acc = rest[2 * nb]
        cid = lax.axis_index("c")
        sid = lax.axis_index("s")
        wid = cid * _NS + sid
        w0 = wid * bmax
        nch = jnp.minimum(bmax, nchunks - w0)

        striped(lambda o, s: zeros_hbm.at[pl.ds(o, s)],
                lambda o, s: acc.at[pl.ds(o, s)], sid)
        plsc.subcore_barrier()

        for ph in range(phases):
            nch_ph = jnp.clip(nch - ph * pb, 0, pb)
            pltpu.sync_copy(src_hbm.at[pl.ds(w0 + ph * pb, pb)], sidx_all)
            pltpu.sync_copy(dst_hbm.at[pl.ds(w0 + ph * pb, pb)], didx_all)

            for b in range(nb):
                @pl.when(b < nch_ph)
                def _(b=b):
                    pltpu.async_copy(table_hbm.at[sidx_all.at[b]],
                                     rows[b], sg[b])

            def step(i, carry):
                base = i * nb
                for b in range(nb):
                    k = base + b

                    @pl.when(k < nch_ph)
                    def _(b=b, k=k):
                        pltpu.make_async_copy(
                            table_hbm.at[sidx_all.at[k]],
                            rows[b], sg[b]).wait()
                        pltpu.sync_copy(rows[b], acc.at[didx_all.at[k]],
                                        add=True)
                    kn = base + nb + b

                    @pl.when(kn < nch_ph)
                    def _(b=b, kn=kn):
                        pltpu.async_copy(table_hbm.at[sidx_all.at[kn]],
                                         rows[b], sg[b])
                return carry

            lax.fori_loop(0, pb // nb, step, 0)

        plsc.subcore_barrier()
        striped(lambda o, s: acc.at[pl.ds(o, s)],
                lambda o, s: out_hbm.at[pl.ds(cid * n + o, s)], sid)

    return functools.partial(
        pl.kernel,
        out_type=jax.ShapeDtypeStruct((_NC * n, w), jnp.float32),
        mesh=plsc.VectorSubcoreMesh(core_axis_name="c", subcore_axis_name="s"),
        compiler_params=pltpu.CompilerParams(use_tc_tiling_on_sc=False)
        if w < 128 else None,
        scratch_types=[
            pltpu.VMEM((pb, _CH), jnp.int32),
            pltpu.VMEM((pb, _CH), jnp.int32),
        ] + [pltpu.VMEM((_CH, w), jnp.float32)] * nb
          + [pltpu.SemaphoreType.DMA] * nb
          + [pltpu.VMEM_SHARED((n, w), jnp.float32)],
    )(body)


def _deg_count(n, e, grp=4):
    nchunks, bmax = _chunk_layout(e)
    striped = _striped(n)
    assert bmax % grp == 0

    def body(dst_hbm, ones_hbm, zeros_hbm, out_hbm, didx_all, rows, *rest):
        sems = rest[:grp]
        acc = rest[grp]
        cid = lax.axis_index("c")
        sid = lax.axis_index("s")
        wid = cid * _NS + sid
        w0 = wid * bmax
        nch = jnp.minimum(bmax, nchunks - w0)

        pltpu.sync_copy(dst_hbm.at[pl.ds(w0, bmax)], didx_all)
        pltpu.sync_copy(ones_hbm, rows)
        striped(lambda o, s: zeros_hbm.at[pl.ds(o, s)],
                lambda o, s: acc.at[pl.ds(o, s)], sid)
        plsc.subcore_barrier()

        def step(i, carry):
            base = i * grp
            sds = [None] * grp
            for b in range(grp):
                k = base + b

                @pl.when(k < nch)
                def _(b=b, k=k):
                    sds[b] = pltpu.make_async_copy(
                        rows, acc.at[didx_all.at[k]], sems[b])
                    sds[b].start(add=True)
            for b in range(grp):
                k = base + b

                @pl.when(k < nch)
                def _(b=b):
                    sds[b].wait()
            return carry

        lax.fori_loop(0, bmax // grp, step, 0)
        plsc.subcore_barrier()
        striped(lambda o, s: acc.at[pl.ds(o, s)],
                lambda o, s: out_hbm.at[pl.ds(cid * n + o, s)], sid)

    return functools.partial(
        pl.kernel,
        out_type=jax.ShapeDtypeStruct((_NC * n, 16), jnp.float32),
        mesh=plsc.VectorSubcoreMesh(core_axis_name="c", subcore_axis_name="s"),
        scratch_types=[
            pltpu.VMEM((bmax, _CH), jnp.int32),
            pltpu.VMEM((_CH, 16), jnp.float32),
        ] + [pltpu.SemaphoreType.DMA] * grp
          + [pltpu.VMEM_SHARED((n, 16), jnp.float32)],
    )(body)


def _dinv_of(da_ref, db_ref):
    deg = da_ref[:, 0:1] + db_ref[:, 0:1] + 1.0
    return lax.rsqrt(deg)


def _tc_prescale(n, bn):
    grid = n // bn

    def body(x_ref, w_ref, da_ref, db_ref, o_ref):
        dinv = _dinv_of(da_ref, db_ref)
        o_ref[...] = jnp.dot(x_ref[...], w_ref[...],
                             preferred_element_type=jnp.float32) * dinv

    return pl.pallas_call(
        body,
        grid=(grid,),
        in_specs=[
            pl.BlockSpec((bn, 128), lambda i: (i, 0)),
            pl.BlockSpec((128, 128), lambda i: (0, 0)),
            pl.BlockSpec((bn, 16), lambda i: (i, 0)),
            pl.BlockSpec((bn, 16), lambda i: (i + grid, 0)),
        ],
        out_specs=pl.BlockSpec((bn, 128), lambda i: (i, 0)),
        out_shape=jax.ShapeDtypeStruct((n, 128), jnp.float32),
    )


def _tc_mid(n, bn):
    grid = n // bn

    def body(pa_ref, pb_ref, hs_ref, da_ref, db_ref, b_ref, w_ref, o_ref):
        dinv = _dinv_of(da_ref, db_ref)
        h = (pa_ref[...] + pb_ref[...] + hs_ref[...]) * dinv + b_ref[...]
        h = jnp.maximum(h, 0.0)
        o_ref[...] = jnp.dot(h, w_ref[...],
                             preferred_element_type=jnp.float32) * dinv

    return pl.pallas_call(
        body,
        grid=(grid,),
        in_specs=[
            pl.BlockSpec((bn, 128), lambda i: (i, 0)),
            pl.BlockSpec((bn, 128), lambda i: (i + grid, 0)),
            pl.BlockSpec((bn, 128), lambda i: (i, 0)),
            pl.BlockSpec((bn, 16), lambda i: (i, 0)),
            pl.BlockSpec((bn, 16), lambda i: (i + grid, 0)),
            pl.BlockSpec((1, 128), lambda i: (0, 0)),
            pl.BlockSpec((128, 128), lambda i: (0, 0)),
        ],
        out_specs=pl.BlockSpec((bn, 128), lambda i: (i, 0)),
        out_shape=jax.ShapeDtypeStruct((n, 128), jnp.float32),
    )


def _tc_decode(n, bn, dnc, dnf):
    grid = n // bn

    def body(pa_ref, pb_ref, hs_ref, da_ref, db_ref, bmu_ref,
             wnc_ref, bnc_ref, wnf_ref, bnf_ref, f_ref, ft_ref):
        dinv = _dinv_of(da_ref, db_ref)
        s = (pa_ref[...] + pb_ref[...] + hs_ref[...])[:, :64]
        mu = s * dinv + bmu_ref[...]
        f_ref[...] = jnp.dot(mu, wnc_ref[...],
                             preferred_element_type=jnp.float32) + bnc_ref[...]
        ft_ref[...] = jnp.dot(mu, wnf_ref[...],
                              preferred_element_type=jnp.float32) + bnf_ref[...]

    return pl.pallas_call(
        body,
        grid=(grid,),
        in_specs=[
            pl.BlockSpec((bn, 128), lambda i: (i, 0)),
            pl.BlockSpec((bn, 128), lambda i: (i + grid, 0)),
            pl.BlockSpec((bn, 128), lambda i: (i, 0)),
            pl.BlockSpec((bn, 16), lambda i: (i, 0)),
            pl.BlockSpec((bn, 16), lambda i: (i + grid, 0)),
            pl.BlockSpec((1, 64), lambda i: (0, 0)),
            pl.BlockSpec((64, dnc), lambda i: (0, 0)),
            pl.BlockSpec((1, dnc), lambda i: (0, 0)),
            pl.BlockSpec((64, dnf), lambda i: (0, 0)),
            pl.BlockSpec((1, dnf), lambda i: (0, 0)),
        ],
        out_specs=[
            pl.BlockSpec((bn, dnc), lambda i: (i, 0)),
            pl.BlockSpec((bn, dnf), lambda i: (i, 0)),
        ],
        out_shape=[
            jax.ShapeDtypeStruct((n, dnc), jnp.float32),
            jax.ShapeDtypeStruct((n, dnf), jnp.float32),
        ],
    )


def kernel(x, edge_index, W1, b1, Wmu, bmu, Wsig, bsig, Wnc, bnc, Wnf, bnf):
    n = x.shape[0]
    e = edge_index.shape[1]
    hid = Wmu.shape[1]
    dnc = Wnc.shape[1]
    dnf = Wnf.shape[1]
    nmax = 40
    bn = 1000

    _, bmax = _chunk_layout(e)
    pad = bmax * _NW * _CH - e
    src2d = jnp.pad(edge_index[0], (0, pad)).reshape(-1, _CH)
    dst2d = jnp.pad(edge_index[1], (0, pad)).reshape(-1, _CH)

    ones16 = jnp.ones((_CH, 16), jnp.float32)
    z16 = jnp.zeros((n, 16), jnp.float32)
    z128 = jnp.zeros((n, 2 * hid), jnp.float32)

    dega = _deg_count(n, e)(dst2d, ones16, z16)
    hs1 = _tc_prescale(n, bn)(x, W1, dega, dega)
    p1 = _seg_sum(n, e, 2 * hid, phases=2)(src2d, dst2d, hs1, z128)
    wmu_pad = jnp.concatenate(
        [Wmu, jnp.zeros((2 * hid, 2 * hid - hid), jnp.float32)], axis=1)
    hs2 = _tc_mid(n, bn)(p1, p1, hs1, dega, dega,
                         b1.reshape(1, -1), wmu_pad)
    p2 = _seg_sum(n, e, 2 * hid, phases=2)(src2d, dst2d, hs2, z128)
    f, ft = _tc_decode(n, bn, dnc, dnf)(p2, p2, hs2, dega, dega,
                                        bmu.reshape(1, -1),
                                        Wnc, bnc.reshape(1, -1),
                                        Wnf, bnf.reshape(1, -1))
    return (f.reshape(n, nmax, dnc // nmax),
            ft.reshape(n, nmax, dnf // nmax))


# speedup vs baseline: 26.6072x; 1.5363x over previous
import functools

import jax
import jax.numpy as jnp
from jax import lax
from jax.experimental import pallas as pl
from jax.experimental.pallas import tpu as pltpu
from jax.experimental.pallas import tpu_sc as plsc

_NC = 2
_NS = 16
_CH = 128


_NW = _NC * _NS


def _chunk_layout(e):
    nchunks = e // _CH
    bmax = (-(-nchunks // _NW) + 7) & ~7
    return nchunks, bmax


def _striped(n):
    rpt = (n // _NS) & ~7
    last = n - (_NS - 1) * rpt

    def copy(mk_src, mk_dst, sid):
        @pl.when(sid < _NS - 1)
        def _():
            pltpu.sync_copy(mk_src(sid * rpt, rpt), mk_dst(sid * rpt, rpt))

        @pl.when(sid == _NS - 1)
        def _():
            pltpu.sync_copy(mk_src((_NS - 1) * rpt, last),
                            mk_dst((_NS - 1) * rpt, last))

    return copy


def _seg_sum(n, e, w, nb=2, phases=1):
    nchunks, bmax = _chunk_layout(e)
    striped = _striped(n)
    pb = bmax // phases
    assert pb % nb == 0 and pb * phases == bmax

    def body(src_hbm, dst_hbm, table_hbm, zeros_hbm, out_hbm,
             sidx_all, didx_all, *rest):
        rows = rest[:nb]
        sg = rest[nb:2 * nb]
        acc = rest[2 * nb]
        cid = lax.axis_index("c")
        sid = lax.axis_index("s")
        wid = cid * _NS + sid
        w0 = wid * bmax
        nch = jnp.minimum(bmax, nchunks - w0)

        striped(lambda o, s: zeros_hbm.at[pl.ds(o, s)],
                lambda o, s: acc.at[pl.ds(o, s)], sid)
        plsc.subcore_barrier()

        for ph in range(phases):
            nch_ph = jnp.clip(nch - ph * pb, 0, pb)
            pltpu.sync_copy(src_hbm.at[pl.ds(w0 + ph * pb, pb)], sidx_all)
            pltpu.sync_copy(dst_hbm.at[pl.ds(w0 + ph * pb, pb)], didx_all)

            for b in range(nb):
                @pl.when(b < nch_ph)
                def _(b=b):
                    pltpu.async_copy(table_hbm.at[sidx_all.at[b]],
                                     rows[b], sg[b])

            def step(i, carry):
                base = i * nb
                for b in range(nb):
                    k = base + b

                    @pl.when(k < nch_ph)
                    def _(b=b, k=k):
                        pltpu.make_async_copy(
                            table_hbm.at[sidx_all.at[k]],
                            rows[b], sg[b]).wait()
                        pltpu.sync_copy(rows[b], acc.at[didx_all.at[k]],
                                        add=True)
                    kn = base + nb + b

                    @pl.when(kn < nch_ph)
                    def _(b=b, kn=kn):
                        pltpu.async_copy(table_hbm.at[sidx_all.at[kn]],
                                         rows[b], sg[b])
                return carry

            lax.fori_loop(0, pb // nb, step, 0)

        plsc.subcore_barrier()
        striped(lambda o, s: acc.at[pl.ds(o, s)],
                lambda o, s: out_hbm.at[pl.ds(cid * n + o, s)], sid)

    return functools.partial(
        pl.kernel,
        out_type=jax.ShapeDtypeStruct((_NC * n, w), jnp.float32),
        mesh=plsc.VectorSubcoreMesh(core_axis_name="c", subcore_axis_name="s"),
        compiler_params=pltpu.CompilerParams(use_tc_tiling_on_sc=False)
        if w < 128 else None,
        scratch_types=[
            pltpu.VMEM((pb, _CH), jnp.int32),
            pltpu.VMEM((pb, _CH), jnp.int32),
        ] + [pltpu.VMEM((_CH, w), jnp.float32)] * nb
          + [pltpu.SemaphoreType.DMA] * nb
          + [pltpu.VMEM_SHARED((n, w), jnp.float32)],
    )(body)


def _deg_count(n, e, grp=4):
    nchunks, bmax = _chunk_layout(e)
    striped = _striped(n)
    assert bmax % grp == 0

    def body(dst_hbm, ones_hbm, zeros_hbm, out_hbm, didx_all, rows, *rest):
        sems = rest[:grp]
        acc = rest[grp]
        cid = lax.axis_index("c")
        sid = lax.axis_index("s")
        wid = cid * _NS + sid
        w0 = wid * bmax
        nch = jnp.minimum(bmax, nchunks - w0)

        pltpu.sync_copy(dst_hbm.at[pl.ds(w0, bmax)], didx_all)
        pltpu.sync_copy(ones_hbm, rows)
        striped(lambda o, s: zeros_hbm.at[pl.ds(o, s)],
                lambda o, s: acc.at[pl.ds(o, s)], sid)
        plsc.subcore_barrier()

        def step(i, carry):
            base = i * grp
            sds = [None] * grp
            for b in range(grp):
                k = base + b

                @pl.when(k < nch)
                def _(b=b, k=k):
                    sds[b] = pltpu.make_async_copy(
                        rows, acc.at[didx_all.at[k]], sems[b])
                    sds[b].start(add=True)
            for b in range(grp):
                k = base + b

                @pl.when(k < nch)
                def _(b=b):
                    sds[b].wait()
            return carry

        lax.fori_loop(0, bmax // grp, step, 0)
        plsc.subcore_barrier()
        striped(lambda o, s: acc.at[pl.ds(o, s)],
                lambda o, s: out_hbm.at[pl.ds(cid * n + o, s)], sid)

    return functools.partial(
        pl.kernel,
        out_type=jax.ShapeDtypeStruct((_NC * n, 16), jnp.float32),
        mesh=plsc.VectorSubcoreMesh(core_axis_name="c", subcore_axis_name="s"),
        scratch_types=[
            pltpu.VMEM((bmax, _CH), jnp.int32),
            pltpu.VMEM((_CH, 16), jnp.float32),
        ] + [pltpu.SemaphoreType.DMA] * grp
          + [pltpu.VMEM_SHARED((n, 16), jnp.float32)],
    )(body)


def _dinv_of(da_ref, db_ref):
    deg = da_ref[:, 0:1] + db_ref[:, 0:1] + 1.0
    return lax.rsqrt(deg)


def _tc_prescale(n, bn):
    grid = n // bn

    def body(x_ref, w_ref, da_ref, db_ref, o_ref):
        dinv = _dinv_of(da_ref, db_ref)
        o_ref[...] = jnp.dot(x_ref[...], w_ref[...],
                             preferred_element_type=jnp.float32) * dinv

    return pl.pallas_call(
        body,
        grid=(grid,),
        in_specs=[
            pl.BlockSpec((bn, 128), lambda i: (i, 0)),
            pl.BlockSpec((128, 128), lambda i: (0, 0)),
            pl.BlockSpec((bn, 16), lambda i: (i, 0)),
            pl.BlockSpec((bn, 16), lambda i: (i + grid, 0)),
        ],
        out_specs=pl.BlockSpec((bn, 128), lambda i: (i, 0)),
        out_shape=jax.ShapeDtypeStruct((n, 128), jnp.float32),
    )


def _tc_mid(n, bn):
    grid = n // bn

    def body(pa_ref, pb_ref, hs_ref, da_ref, db_ref, b_ref, w_ref, o_ref):
        dinv = _dinv_of(da_ref, db_ref)
        h = (pa_ref[...] + pb_ref[...] + hs_ref[...]) * dinv + b_ref[...]
        h = jnp.maximum(h, 0.0)
        o_ref[...] = jnp.dot(h, w_ref[...],
                             preferred_element_type=jnp.float32) * dinv

    return pl.pallas_call(
        body,
        grid=(grid,),
        in_specs=[
            pl.BlockSpec((bn, 128), lambda i: (i, 0)),
            pl.BlockSpec((bn, 128), lambda i: (i + grid, 0)),
            pl.BlockSpec((bn, 128), lambda i: (i, 0)),
            pl.BlockSpec((bn, 16), lambda i: (i, 0)),
            pl.BlockSpec((bn, 16), lambda i: (i + grid, 0)),
            pl.BlockSpec((1, 128), lambda i: (0, 0)),
            pl.BlockSpec((128, 128), lambda i: (0, 0)),
        ],
        out_specs=pl.BlockSpec((bn, 128), lambda i: (i, 0)),
        out_shape=jax.ShapeDtypeStruct((n, 128), jnp.float32),
    )


def _tc_mu(n, bn):
    grid = n // bn

    def body(pa_ref, pb_ref, hs_ref, da_ref, db_ref, bmu_ref, o_ref):
        dinv = _dinv_of(da_ref, db_ref)
        s = (pa_ref[...] + pb_ref[...] + hs_ref[...])[:, :64]
        o_ref[...] = s * dinv + bmu_ref[...]

    return pl.pallas_call(
        body,
        grid=(grid,),
        in_specs=[
            pl.BlockSpec((bn, 128), lambda i: (i, 0)),
            pl.BlockSpec((bn, 128), lambda i: (i + grid, 0)),
            pl.BlockSpec((bn, 128), lambda i: (i, 0)),
            pl.BlockSpec((bn, 16), lambda i: (i, 0)),
            pl.BlockSpec((bn, 16), lambda i: (i + grid, 0)),
            pl.BlockSpec((1, 64), lambda i: (0, 0)),
        ],
        out_specs=pl.BlockSpec((bn, 64), lambda i: (i, 0)),
        out_shape=jax.ShapeDtypeStruct((n, 64), jnp.float32),
    )


def _tc_decode_t(n, rows, br):
    grid = rows // br

    def body(w_ref, b_ref, mu_ref, o_ref):
        g = lax.dot_general(w_ref[...], mu_ref[...],
                            (((1,), (1,)), ((), ())),
                            preferred_element_type=jnp.float32)
        o_ref[...] = g + b_ref[...]

    return pl.pallas_call(
        body,
        grid=(grid,),
        in_specs=[
            pl.BlockSpec((br, 64), lambda i: (i, 0)),
            pl.BlockSpec((br, 1), lambda i: (i, 0)),
            pl.BlockSpec((n, 64), lambda i: (0, 0)),
        ],
        out_specs=pl.BlockSpec((br, n), lambda i: (i, 0)),
        out_shape=jax.ShapeDtypeStruct((rows, n), jnp.float32),
    )


def kernel(x, edge_index, W1, b1, Wmu, bmu, Wsig, bsig, Wnc, bnc, Wnf, bnf):
    n = x.shape[0]
    e = edge_index.shape[1]
    hid = Wmu.shape[1]
    dnc = Wnc.shape[1]
    dnf = Wnf.shape[1]
    nmax = 40
    bn = 1000

    _, bmax = _chunk_layout(e)
    pad = bmax * _NW * _CH - e
    src2d = jnp.pad(edge_index[0], (0, pad)).reshape(-1, _CH)
    dst2d = jnp.pad(edge_index[1], (0, pad)).reshape(-1, _CH)

    ones16 = jnp.ones((_CH, 16), jnp.float32)
    z16 = jnp.zeros((n, 16), jnp.float32)
    z128 = jnp.zeros((n, 2 * hid), jnp.float32)

    dega = _deg_count(n, e)(dst2d, ones16, z16)
    hs1 = _tc_prescale(n, bn)(x, W1, dega, dega)
    p1 = _seg_sum(n, e, 2 * hid, phases=2)(src2d, dst2d, hs1, z128)
    wmu_pad = jnp.concatenate(
        [Wmu, jnp.zeros((2 * hid, 2 * hid - hid), jnp.float32)], axis=1)
    hs2 = _tc_mid(n, bn)(p1, p1, hs1, dega, dega,
                         b1.reshape(1, -1), wmu_pad)
    p2 = _seg_sum(n, e, 2 * hid, phases=2)(src2d, dst2d, hs2, z128)
    mu = _tc_mu(n, bn)(p2, p2, hs2, dega, dega, bmu.reshape(1, -1))

    ncls = dnc // nmax
    wncp = Wnc.reshape(-1, nmax, ncls).transpose(2, 1, 0).reshape(dnc, -1)
    bncp = bnc.reshape(nmax, ncls).T.reshape(dnc, 1)
    g = _tc_decode_t(n, dnc, 240)(wncp, bncp, mu)
    h = _tc_decode_t(n, dnf, 256)(Wnf.T, bnf.reshape(dnf, 1), mu)
    f_hat = g.reshape(ncls, nmax, n).transpose(2, 1, 0)
    feat_hat = h.reshape(nmax, dnf // nmax, n).transpose(2, 0, 1)
    return (f_hat, feat_hat)
